# baseline (speedup 1.0000x reference)
import functools

import jax
import jax.numpy as jnp
from jax import lax
from jax.experimental import pallas as pl
from jax.experimental.pallas import tpu as pltpu
from jax.experimental.pallas import tpu_sc as plsc

N = 10000
E = 320000
D = 128
H = 64
C = 64

NC = 2
NS = 16
NW = NC * NS
CH = 128
NCHUNK = 80
EPW = NCHUNK * CH
EPAD = NW * EPW
NPAD = 10240
RPT = NPAD // NS


def _seg_sum_sc(feat, src3, dst3, zeros):
    F = feat.shape[1]
    mesh = plsc.VectorSubcoreMesh(core_axis_name="c", subcore_axis_name="s")

    @functools.partial(
        pl.kernel,
        mesh=mesh,
        compiler_params=pltpu.CompilerParams(use_tc_tiling_on_sc=False),
        out_type=jax.ShapeDtypeStruct((NC * NPAD, F), jnp.float32),
        scratch_types=[
            pltpu.VMEM((NCHUNK, CH), jnp.int32),
            pltpu.VMEM((NCHUNK, CH), jnp.int32),
            pltpu.VMEM((CH, F), jnp.float32),
            pltpu.VMEM((CH, F), jnp.float32),
            pltpu.VMEM_SHARED((NPAD, F), jnp.float32),
            pltpu.SemaphoreType.DMA,
            pltpu.SemaphoreType.DMA,
        ],
    )
    def k(feat_hbm, src_hbm, dst_hbm, zero_hbm, out_hbm, src_v, dst_v,
          buf_a, buf_b, agg, sem_a, sem_b):
        cid = lax.axis_index("c")
        sid = lax.axis_index("s")
        wid = sid * NC + cid
        pltpu.sync_copy(src_hbm.at[wid], src_v)
        pltpu.sync_copy(dst_hbm.at[wid], dst_v)
        pltpu.sync_copy(zero_hbm.at[pl.ds(sid * RPT, RPT)],
                        agg.at[pl.ds(sid * RPT, RPT)])
        plsc.subcore_barrier()

        pltpu.async_copy(feat_hbm.at[src_v.at[0]], buf_a, sem_a)
        pltpu.async_copy(feat_hbm.at[src_v.at[1]], buf_b, sem_b)

        @pl.loop(0, NCHUNK - 2, step=2)
        def _(j):
            pltpu.make_async_copy(feat_hbm.at[src_v.at[j]], buf_a, sem_a).wait()
            pltpu.sync_copy(buf_a, agg.at[dst_v.at[j]], add=True)
            pltpu.async_copy(feat_hbm.at[src_v.at[j + 2]], buf_a, sem_a)
            pltpu.make_async_copy(
                feat_hbm.at[src_v.at[j + 1]], buf_b, sem_b).wait()
            pltpu.sync_copy(buf_b, agg.at[dst_v.at[j + 1]], add=True)
            pltpu.async_copy(feat_hbm.at[src_v.at[j + 3]], buf_b, sem_b)

        pltpu.make_async_copy(
            feat_hbm.at[src_v.at[NCHUNK - 2]], buf_a, sem_a).wait()
        pltpu.sync_copy(buf_a, agg.at[dst_v.at[NCHUNK - 2]], add=True)
        pltpu.make_async_copy(
            feat_hbm.at[src_v.at[NCHUNK - 1]], buf_b, sem_b).wait()
        pltpu.sync_copy(buf_b, agg.at[dst_v.at[NCHUNK - 1]], add=True)

        plsc.subcore_barrier()
        pltpu.sync_copy(agg.at[pl.ds(sid * RPT, RPT)],
                        out_hbm.at[pl.ds(cid * NPAD + sid * RPT, RPT)])

    return k(feat, src3, dst3, zeros)


def _dot_t(a, w):
    return lax.dot_general(a, w, (((1,), (1,)), ((), ())),
                           precision=lax.Precision.HIGHEST,
                           preferred_element_type=jnp.float32)


def _tc1_body(x_ref, wrel_ref, wroot_ref, b_ref, xr_ref, xroot_ref):
    x = x_ref[...]
    xr_ref[...] = _dot_t(x, wrel_ref[...])
    xroot_ref[...] = _dot_t(x, wroot_ref[...]) + b_ref[...]


def _tc2_body(part_ref, xroot_ref, wrel_ref, wroot_ref, b_ref, hr_ref,
              hroot_ref):
    h = jnp.maximum(part_ref[0] + part_ref[1] + xroot_ref[...], 0.0)
    hr_ref[...] = _dot_t(h, wrel_ref[...])
    hroot_ref[...] = _dot_t(h, wroot_ref[...]) + b_ref[...]


def _tc3_body(part_ref, hroot_ref, out_ref):
    z = part_ref[0] + part_ref[1] + hroot_ref[...]
    zmax = jnp.max(z, axis=1, keepdims=True)
    s = z - zmax
    out_ref[...] = s - jnp.log(jnp.sum(jnp.exp(s), axis=1, keepdims=True))


def kernel(x, edge_index, W1_rel, b1, W1_root, W2_rel, b2, W2_root):
    xp = jnp.concatenate(
        [x, jnp.zeros((NPAD - N, D), jnp.float32)], axis=0)
    epad = jnp.concatenate(
        [edge_index, jnp.full((2, EPAD - E), N, jnp.int32)], axis=1)
    src3 = epad[0].reshape(NW, NCHUNK, CH)
    dst3 = epad[1].reshape(NW, NCHUNK, CH)
    zeros = jnp.zeros((NPAD, H), jnp.float32)
    b1r = b1.reshape(1, H)
    b2r = b2.reshape(1, C)

    xr, xroot = pl.pallas_call(
        _tc1_body,
        out_shape=[jax.ShapeDtypeStruct((NPAD, H), jnp.float32),
                   jax.ShapeDtypeStruct((NPAD, H), jnp.float32)],
    )(xp, W1_rel, W1_root, b1r)

    part1 = _seg_sum_sc(xr, src3, dst3, zeros).reshape(NC, NPAD, H)

    hr, hroot = pl.pallas_call(
        _tc2_body,
        out_shape=[jax.ShapeDtypeStruct((NPAD, C), jnp.float32),
                   jax.ShapeDtypeStruct((NPAD, C), jnp.float32)],
    )(part1, xroot, W2_rel, W2_root, b2r)

    part2 = _seg_sum_sc(hr, src3, dst3, zeros).reshape(NC, NPAD, C)

    out = pl.pallas_call(
        _tc3_body,
        out_shape=jax.ShapeDtypeStruct((NPAD, C), jnp.float32),
    )(part2, hroot)

    return out[:N]


# speedup vs baseline: 1.0061x; 1.0061x over previous
import functools

import jax
import jax.numpy as jnp
from jax import lax
from jax.experimental import pallas as pl
from jax.experimental.pallas import tpu as pltpu
from jax.experimental.pallas import tpu_sc as plsc

N = 10000
E = 320000
D = 128
H = 64
C = 64

NC = 2
NS = 16
NW = NC * NS
CH = 80
NCHUNK = 128
EPW = NCHUNK * CH
EPAD = NW * EPW
NPAD = 10240
RPT = NPAD // NS


def _seg_sum_sc(feat, src3, dst3, zeros):
    F = feat.shape[1]
    mesh = plsc.VectorSubcoreMesh(core_axis_name="c", subcore_axis_name="s")

    @functools.partial(
        pl.kernel,
        mesh=mesh,
        compiler_params=pltpu.CompilerParams(use_tc_tiling_on_sc=False),
        out_type=jax.ShapeDtypeStruct((NC * NPAD, F), jnp.float32),
        scratch_types=[
            pltpu.VMEM((NCHUNK, CH), jnp.int32),
            pltpu.VMEM((NCHUNK, CH), jnp.int32),
            pltpu.VMEM((CH, F), jnp.float32),
            pltpu.VMEM((CH, F), jnp.float32),
            pltpu.VMEM_SHARED((NPAD, F), jnp.float32),
            pltpu.SemaphoreType.DMA,
            pltpu.SemaphoreType.DMA,
        ],
    )
    def k(feat_hbm, src_hbm, dst_hbm, zero_hbm, out_hbm, src_v, dst_v,
          buf_a, buf_b, agg, sem_a, sem_b):
        cid = lax.axis_index("c")
        sid = lax.axis_index("s")
        wid = sid * NC + cid
        pltpu.sync_copy(src_hbm.at[wid], src_v)
        pltpu.sync_copy(dst_hbm.at[wid], dst_v)
        pltpu.sync_copy(zero_hbm.at[pl.ds(sid * RPT, RPT)],
                        agg.at[pl.ds(sid * RPT, RPT)])
        plsc.subcore_barrier()

        pltpu.async_copy(feat_hbm.at[src_v.at[0]], buf_a, sem_a)
        pltpu.async_copy(feat_hbm.at[src_v.at[1]], buf_b, sem_b)

        @pl.loop(0, NCHUNK - 2, step=2)
        def _(j):
            pltpu.make_async_copy(feat_hbm.at[src_v.at[j]], buf_a, sem_a).wait()
            pltpu.sync_copy(buf_a, agg.at[dst_v.at[j]], add=True)
            pltpu.async_copy(feat_hbm.at[src_v.at[j + 2]], buf_a, sem_a)
            pltpu.make_async_copy(
                feat_hbm.at[src_v.at[j + 1]], buf_b, sem_b).wait()
            pltpu.sync_copy(buf_b, agg.at[dst_v.at[j + 1]], add=True)
            pltpu.async_copy(feat_hbm.at[src_v.at[j + 3]], buf_b, sem_b)

        pltpu.make_async_copy(
            feat_hbm.at[src_v.at[NCHUNK - 2]], buf_a, sem_a).wait()
        pltpu.sync_copy(buf_a, agg.at[dst_v.at[NCHUNK - 2]], add=True)
        pltpu.make_async_copy(
            feat_hbm.at[src_v.at[NCHUNK - 1]], buf_b, sem_b).wait()
        pltpu.sync_copy(buf_b, agg.at[dst_v.at[NCHUNK - 1]], add=True)

        plsc.subcore_barrier()
        pltpu.sync_copy(agg.at[pl.ds(sid * RPT, RPT)],
                        out_hbm.at[pl.ds(cid * NPAD + sid * RPT, RPT)])

    return k(feat, src3, dst3, zeros)


def _dot_t(a, w):
    return lax.dot_general(a, w, (((1,), (1,)), ((), ())),
                           precision=lax.Precision.HIGHEST,
                           preferred_element_type=jnp.float32)


def _tc1_body(x_ref, wrel_ref, wroot_ref, b_ref, xr_ref, xroot_ref):
    x = x_ref[...]
    xr_ref[...] = _dot_t(x, wrel_ref[...])
    xroot_ref[...] = _dot_t(x, wroot_ref[...]) + b_ref[...]


def _tc2_body(part_ref, xroot_ref, wrel_ref, wroot_ref, b_ref, hr_ref,
              hroot_ref):
    h = jnp.maximum(part_ref[0] + part_ref[1] + xroot_ref[...], 0.0)
    hr_ref[...] = _dot_t(h, wrel_ref[...])
    hroot_ref[...] = _dot_t(h, wroot_ref[...]) + b_ref[...]


def _tc3_body(part_ref, hroot_ref, out_ref):
    z = part_ref[0] + part_ref[1] + hroot_ref[...]
    zmax = jnp.max(z, axis=1, keepdims=True)
    s = z - zmax
    out_ref[...] = s - jnp.log(jnp.sum(jnp.exp(s), axis=1, keepdims=True))


def kernel(x, edge_index, W1_rel, b1, W1_root, W2_rel, b2, W2_root):
    xp = jnp.concatenate(
        [x, jnp.zeros((NPAD - N, D), jnp.float32)], axis=0)
    epad = jnp.concatenate(
        [edge_index, jnp.full((2, EPAD - E), N, jnp.int32)], axis=1)
    src3 = epad[0].reshape(NW, NCHUNK, CH)
    dst3 = epad[1].reshape(NW, NCHUNK, CH)
    zeros = jnp.zeros((NPAD, H), jnp.float32)
    b1r = b1.reshape(1, H)
    b2r = b2.reshape(1, C)

    xr, xroot = pl.pallas_call(
        _tc1_body,
        out_shape=[jax.ShapeDtypeStruct((NPAD, H), jnp.float32),
                   jax.ShapeDtypeStruct((NPAD, H), jnp.float32)],
    )(xp, W1_rel, W1_root, b1r)

    part1 = _seg_sum_sc(xr, src3, dst3, zeros).reshape(NC, NPAD, H)

    hr, hroot = pl.pallas_call(
        _tc2_body,
        out_shape=[jax.ShapeDtypeStruct((NPAD, C), jnp.float32),
                   jax.ShapeDtypeStruct((NPAD, C), jnp.float32)],
    )(part1, xroot, W2_rel, W2_root, b2r)

    part2 = _seg_sum_sc(hr, src3, dst3, zeros).reshape(NC, NPAD, C)

    out = pl.pallas_call(
        _tc3_body,
        out_shape=jax.ShapeDtypeStruct((NPAD, C), jnp.float32),
    )(part2, hroot)

    return out[:N]


# speedup vs baseline: 1.4252x; 1.4166x over previous
import functools

import jax
import jax.numpy as jnp
from jax import lax
from jax.experimental import pallas as pl
from jax.experimental.pallas import tpu as pltpu
from jax.experimental.pallas import tpu_sc as plsc

N = 10000
E = 320000
D = 128
H = 64
C = 64

NC = 2
NS = 16
NW = NC * NS
EPW = E // NW
CH = 80
NCHUNK = EPW // CH
RPT = N // NS


def _seg_sum_sc(feat, src3, dst3, zeros):
    F = feat.shape[1]
    mesh = plsc.VectorSubcoreMesh(core_axis_name="c", subcore_axis_name="s")

    @functools.partial(
        pl.kernel,
        mesh=mesh,
        compiler_params=pltpu.CompilerParams(use_tc_tiling_on_sc=False),
        out_type=jax.ShapeDtypeStruct((NC * N, F), jnp.float32),
        scratch_types=[
            pltpu.VMEM((NCHUNK, CH), jnp.int32),
            pltpu.VMEM((NCHUNK, CH), jnp.int32),
            pltpu.VMEM((CH, F), jnp.float32),
            pltpu.VMEM_SHARED((N, F), jnp.float32),
        ],
    )
    def k(feat_hbm, src_hbm, dst_hbm, zero_hbm, out_hbm, src_v, dst_v, buf, agg):
        cid = lax.axis_index("c")
        sid = lax.axis_index("s")
        wid = sid * NC + cid
        pltpu.sync_copy(src_hbm.at[wid], src_v)
        pltpu.sync_copy(dst_hbm.at[wid], dst_v)
        pltpu.sync_copy(zero_hbm.at[pl.ds(sid * RPT, RPT)],
                        agg.at[pl.ds(sid * RPT, RPT)])
        plsc.subcore_barrier()

        @pl.loop(0, NCHUNK)
        def _(j):
            pltpu.sync_copy(feat_hbm.at[src_v.at[j]], buf)
            pltpu.sync_copy(buf, agg.at[dst_v.at[j]], add=True)

        plsc.subcore_barrier()
        pltpu.sync_copy(agg.at[pl.ds(sid * RPT, RPT)],
                        out_hbm.at[pl.ds(cid * N + sid * RPT, RPT)])

    return k(feat, src3, dst3, zeros)


def _dot_t(a, w):
    return lax.dot_general(a, w, (((1,), (1,)), ((), ())),
                           precision=lax.Precision.HIGHEST,
                           preferred_element_type=jnp.float32)


def _tc1_body(x_ref, wrel_ref, wroot_ref, b_ref, xr_ref, xroot_ref):
    x = x_ref[...]
    xr_ref[...] = _dot_t(x, wrel_ref[...])
    xroot_ref[...] = _dot_t(x, wroot_ref[...]) + b_ref[...]


def _tc2_body(part_ref, xroot_ref, wrel_ref, wroot_ref, b_ref, hr_ref,
              hroot_ref):
    h = jnp.maximum(part_ref[0] + part_ref[1] + xroot_ref[...], 0.0)
    hr_ref[...] = _dot_t(h, wrel_ref[...])
    hroot_ref[...] = _dot_t(h, wroot_ref[...]) + b_ref[...]


def _tc3_body(part_ref, hroot_ref, out_ref):
    z = part_ref[0] + part_ref[1] + hroot_ref[...]
    zmax = jnp.max(z, axis=1, keepdims=True)
    s = z - zmax
    out_ref[...] = s - jnp.log(jnp.sum(jnp.exp(s), axis=1, keepdims=True))


def kernel(x, edge_index, W1_rel, b1, W1_root, W2_rel, b2, W2_root):
    src3 = edge_index[0].reshape(NW, NCHUNK, CH)
    dst3 = edge_index[1].reshape(NW, NCHUNK, CH)
    zeros = jnp.zeros((N, H), jnp.float32)
    b1r = b1.reshape(1, H)
    b2r = b2.reshape(1, C)

    xr, xroot = pl.pallas_call(
        _tc1_body,
        out_shape=[jax.ShapeDtypeStruct((N, H), jnp.float32),
                   jax.ShapeDtypeStruct((N, H), jnp.float32)],
    )(x, W1_rel, W1_root, b1r)

    part1 = _seg_sum_sc(xr, src3, dst3, zeros).reshape(NC, N, H)

    hr, hroot = pl.pallas_call(
        _tc2_body,
        out_shape=[jax.ShapeDtypeStruct((N, C), jnp.float32),
                   jax.ShapeDtypeStruct((N, C), jnp.float32)],
    )(part1, xroot, W2_rel, W2_root, b2r)

    part2 = _seg_sum_sc(hr, src3, dst3, zeros).reshape(NC, N, C)

    out = pl.pallas_call(
        _tc3_body,
        out_shape=jax.ShapeDtypeStruct((N, C), jnp.float32),
    )(part2, hroot)

    return out


# speedup vs baseline: 1.6600x; 1.1647x over previous
import functools

import jax
import jax.numpy as jnp
from jax import lax
from jax.experimental import pallas as pl
from jax.experimental.pallas import tpu as pltpu
from jax.experimental.pallas import tpu_sc as plsc

N = 10000
E = 320000
D = 128
H = 64
C = 64

NC = 2
NS = 16
NW = NC * NS
CH = 128
NCHUNK = 80
EPW = NCHUNK * CH
EPAD = NW * EPW
RPT = N // NS
NACC = N + 16


def _seg_sum_sc(feat, src3, dst3, zeros):
    F = feat.shape[1]
    mesh = plsc.VectorSubcoreMesh(core_axis_name="c", subcore_axis_name="s")

    @functools.partial(
        pl.kernel,
        mesh=mesh,
        compiler_params=pltpu.CompilerParams(use_tc_tiling_on_sc=False),
        out_type=jax.ShapeDtypeStruct((NC * N, F), jnp.float32),
        scratch_types=[
            pltpu.VMEM((NCHUNK, CH), jnp.int32),
            pltpu.VMEM((NCHUNK, CH), jnp.int32),
            pltpu.VMEM((CH, F), jnp.float32),
            pltpu.VMEM_SHARED((NACC, F), jnp.float32),
        ],
    )
    def k(feat_hbm, src_hbm, dst_hbm, zero_hbm, out_hbm, src_v, dst_v, buf, agg):
        cid = lax.axis_index("c")
        sid = lax.axis_index("s")
        wid = sid * NC + cid
        pltpu.sync_copy(src_hbm.at[wid], src_v)
        pltpu.sync_copy(dst_hbm.at[wid], dst_v)
        pltpu.sync_copy(zero_hbm.at[pl.ds(sid * RPT, RPT)],
                        agg.at[pl.ds(sid * RPT, RPT)])
        plsc.subcore_barrier()

        @pl.loop(0, NCHUNK)
        def _(j):
            pltpu.sync_copy(feat_hbm.at[src_v.at[j]], buf)
            pltpu.sync_copy(buf, agg.at[dst_v.at[j]], add=True)

        plsc.subcore_barrier()
        pltpu.sync_copy(agg.at[pl.ds(sid * RPT, RPT)],
                        out_hbm.at[pl.ds(cid * N + sid * RPT, RPT)])

    return k(feat, src3, dst3, zeros)


def _dot_t(a, w):
    return lax.dot_general(a, w, (((1,), (1,)), ((), ())),
                           precision=lax.Precision.HIGHEST,
                           preferred_element_type=jnp.float32)


def _tc1_body(x_ref, wrel_ref, wroot_ref, b_ref, xr_ref, xroot_ref):
    x = x_ref[...]
    xr_ref[...] = _dot_t(x, wrel_ref[...])
    xroot_ref[...] = _dot_t(x, wroot_ref[...]) + b_ref[...]


def _tc2_body(part_ref, xroot_ref, wrel_ref, wroot_ref, b_ref, hr_ref,
              hroot_ref):
    h = jnp.maximum(part_ref[0] + part_ref[1] + xroot_ref[...], 0.0)
    hr_ref[...] = _dot_t(h, wrel_ref[...])
    hroot_ref[...] = _dot_t(h, wroot_ref[...]) + b_ref[...]


def _tc3_body(part_ref, hroot_ref, out_ref):
    z = part_ref[0] + part_ref[1] + hroot_ref[...]
    zmax = jnp.max(z, axis=1, keepdims=True)
    s = z - zmax
    out_ref[...] = s - jnp.log(jnp.sum(jnp.exp(s), axis=1, keepdims=True))


def kernel(x, edge_index, W1_rel, b1, W1_root, W2_rel, b2, W2_root):
    pad_cnt = EPAD - E
    ar = jnp.arange(pad_cnt, dtype=jnp.int32)
    pad_src = (ar * 131) % N
    pad_dst = N + (ar % 16)
    src3 = jnp.concatenate([edge_index[0], pad_src]).reshape(NW, NCHUNK, CH)
    dst3 = jnp.concatenate([edge_index[1], pad_dst]).reshape(NW, NCHUNK, CH)
    zeros = jnp.zeros((N, H), jnp.float32)
    b1r = b1.reshape(1, H)
    b2r = b2.reshape(1, C)

    xr, xroot = pl.pallas_call(
        _tc1_body,
        out_shape=[jax.ShapeDtypeStruct((N, H), jnp.float32),
                   jax.ShapeDtypeStruct((N, H), jnp.float32)],
    )(x, W1_rel, W1_root, b1r)

    part1 = _seg_sum_sc(xr, src3, dst3, zeros).reshape(NC, N, H)

    hr, hroot = pl.pallas_call(
        _tc2_body,
        out_shape=[jax.ShapeDtypeStruct((N, C), jnp.float32),
                   jax.ShapeDtypeStruct((N, C), jnp.float32)],
    )(part1, xroot, W2_rel, W2_root, b2r)

    part2 = _seg_sum_sc(hr, src3, dst3, zeros).reshape(NC, N, C)

    out = pl.pallas_call(
        _tc3_body,
        out_shape=jax.ShapeDtypeStruct((N, C), jnp.float32),
    )(part2, hroot)

    return out


# speedup vs baseline: 1.7181x; 1.0350x over previous
import functools

import jax
import jax.numpy as jnp
from jax import lax
from jax.experimental import pallas as pl
from jax.experimental.pallas import tpu as pltpu
from jax.experimental.pallas import tpu_sc as plsc

N = 10000
E = 320000
D = 128
H = 64
C = 64

NC = 2
NS = 16
NW = NC * NS
CH = 128
NCHUNK = 80
EPW = NCHUNK * CH
EPAD = NW * EPW
RPT = N // NS
NACC = N + 16


def _seg_sum_sc(feat, src3, dst3, zeros):
    F = feat.shape[1]
    mesh = plsc.VectorSubcoreMesh(core_axis_name="c", subcore_axis_name="s")

    @functools.partial(
        pl.kernel,
        mesh=mesh,
        compiler_params=pltpu.CompilerParams(use_tc_tiling_on_sc=False),
        out_type=jax.ShapeDtypeStruct((NC * N, F), jnp.float32),
        scratch_types=[
            pltpu.VMEM((NCHUNK, CH), jnp.int32),
            pltpu.VMEM((NCHUNK, CH), jnp.int32),
            pltpu.VMEM((CH, F), jnp.float32),
            pltpu.VMEM_SHARED((NACC, F), jnp.float32),
            pltpu.VMEM_SHARED((N, F), jnp.float32),
        ],
    )
    def k(feat_hbm, src_hbm, dst_hbm, zero_hbm, out_hbm, src_v, dst_v, buf,
          agg, feat_s):
        cid = lax.axis_index("c")
        sid = lax.axis_index("s")
        wid = sid * NC + cid
        pltpu.sync_copy(src_hbm.at[wid], src_v)
        pltpu.sync_copy(dst_hbm.at[wid], dst_v)
        pltpu.sync_copy(zero_hbm.at[pl.ds(sid * RPT, RPT)],
                        agg.at[pl.ds(sid * RPT, RPT)])
        pltpu.sync_copy(feat_hbm.at[pl.ds(sid * RPT, RPT)],
                        feat_s.at[pl.ds(sid * RPT, RPT)])
        plsc.subcore_barrier()

        @pl.loop(0, NCHUNK)
        def _(j):
            pltpu.sync_copy(feat_s.at[src_v.at[j]], buf)
            pltpu.sync_copy(buf, agg.at[dst_v.at[j]], add=True)

        plsc.subcore_barrier()
        pltpu.sync_copy(agg.at[pl.ds(sid * RPT, RPT)],
                        out_hbm.at[pl.ds(cid * N + sid * RPT, RPT)])

    return k(feat, src3, dst3, zeros)


def _dot_t(a, w):
    return lax.dot_general(a, w, (((1,), (1,)), ((), ())),
                           precision=lax.Precision.HIGHEST,
                           preferred_element_type=jnp.float32)


def _tc1_body(x_ref, wrel_ref, wroot_ref, b_ref, xr_ref, xroot_ref):
    x = x_ref[...]
    xr_ref[...] = _dot_t(x, wrel_ref[...])
    xroot_ref[...] = _dot_t(x, wroot_ref[...]) + b_ref[...]


def _tc2_body(part_ref, xroot_ref, wrel_ref, wroot_ref, b_ref, hr_ref,
              hroot_ref):
    h = jnp.maximum(part_ref[0] + part_ref[1] + xroot_ref[...], 0.0)
    hr_ref[...] = _dot_t(h, wrel_ref[...])
    hroot_ref[...] = _dot_t(h, wroot_ref[...]) + b_ref[...]


def _tc3_body(part_ref, hroot_ref, out_ref):
    z = part_ref[0] + part_ref[1] + hroot_ref[...]
    zmax = jnp.max(z, axis=1, keepdims=True)
    s = z - zmax
    out_ref[...] = s - jnp.log(jnp.sum(jnp.exp(s), axis=1, keepdims=True))


def kernel(x, edge_index, W1_rel, b1, W1_root, W2_rel, b2, W2_root):
    pad_cnt = EPAD - E
    ar = jnp.arange(pad_cnt, dtype=jnp.int32)
    pad_src = (ar * 131) % N
    pad_dst = N + (ar % 16)
    src3 = jnp.concatenate([edge_index[0], pad_src]).reshape(NW, NCHUNK, CH)
    dst3 = jnp.concatenate([edge_index[1], pad_dst]).reshape(NW, NCHUNK, CH)
    zeros = jnp.zeros((N, H), jnp.float32)
    b1r = b1.reshape(1, H)
    b2r = b2.reshape(1, C)

    xr, xroot = pl.pallas_call(
        _tc1_body,
        out_shape=[jax.ShapeDtypeStruct((N, H), jnp.float32),
                   jax.ShapeDtypeStruct((N, H), jnp.float32)],
    )(x, W1_rel, W1_root, b1r)

    part1 = _seg_sum_sc(xr, src3, dst3, zeros).reshape(NC, N, H)

    hr, hroot = pl.pallas_call(
        _tc2_body,
        out_shape=[jax.ShapeDtypeStruct((N, C), jnp.float32),
                   jax.ShapeDtypeStruct((N, C), jnp.float32)],
    )(part1, xroot, W2_rel, W2_root, b2r)

    part2 = _seg_sum_sc(hr, src3, dst3, zeros).reshape(NC, N, C)

    out = pl.pallas_call(
        _tc3_body,
        out_shape=jax.ShapeDtypeStruct((N, C), jnp.float32),
    )(part2, hroot)

    return out


# speedup vs baseline: 2.5663x; 1.4937x over previous
import functools

import jax
import jax.numpy as jnp
from jax import lax
from jax.experimental import pallas as pl
from jax.experimental.pallas import tpu as pltpu
from jax.experimental.pallas import tpu_sc as plsc

N = 10000
E = 320000
D = 128
H = 64
C = 64

NC = 2
NS = 16
NW = NC * NS
CH = 128
NCHUNK = 80
G = 8
NGRP = NCHUNK // G
EPW = NCHUNK * CH
EPAD = NW * EPW
RPT = N // NS
NACC = N + 16


def _seg_sum_sc(feat, src3, dst3, zeros):
    F = feat.shape[1]
    mesh = plsc.VectorSubcoreMesh(core_axis_name="c", subcore_axis_name="s")

    @functools.partial(
        pl.kernel,
        mesh=mesh,
        compiler_params=pltpu.CompilerParams(use_tc_tiling_on_sc=False),
        out_type=jax.ShapeDtypeStruct((NC * N, F), jnp.float32),
        scratch_types=[
            pltpu.VMEM((NCHUNK, CH), jnp.int32),
            pltpu.VMEM((NCHUNK, CH), jnp.int32),
            [pltpu.VMEM((CH, F), jnp.float32) for _ in range(G)],
            pltpu.VMEM_SHARED((NACC, F), jnp.float32),
            pltpu.SemaphoreType.DMA,
            pltpu.SemaphoreType.DMA,
        ],
    )
    def k(feat_hbm, src_hbm, dst_hbm, zero_hbm, out_hbm, src_v, dst_v, bufs,
          agg, sem_g, sem_s):
        cid = lax.axis_index("c")
        sid = lax.axis_index("s")
        wid = sid * NC + cid
        pltpu.sync_copy(src_hbm.at[wid], src_v)
        pltpu.sync_copy(dst_hbm.at[wid], dst_v)
        pltpu.sync_copy(zero_hbm.at[pl.ds(sid * RPT, RPT)],
                        agg.at[pl.ds(sid * RPT, RPT)])
        plsc.subcore_barrier()

        def gather_start(j, b):
            pltpu.async_copy(feat_hbm.at[src_v.at[j]], bufs[b], sem_g)

        def gather_wait(j, b):
            pltpu.make_async_copy(feat_hbm.at[src_v.at[j]], bufs[b], sem_g).wait()

        def scatter_start(j, b):
            pltpu.async_copy(bufs[b], agg.at[dst_v.at[j]], sem_s, add=True)

        def scatter_wait(j, b):
            pltpu.make_async_copy(bufs[b], agg.at[dst_v.at[j]], sem_s).wait()

        for b in range(G):
            gather_start(b, b)

        @pl.loop(0, NGRP - 1)
        def _(g):
            base = g * G
            for b in range(G):
                gather_wait(base + b, b)
                scatter_start(base + b, b)
            for b in range(G):
                scatter_wait(base + b, b)
                gather_start(base + G + b, b)

        base = (NGRP - 1) * G
        for b in range(G):
            gather_wait(base + b, b)
            scatter_start(base + b, b)
        for b in range(G):
            scatter_wait(base + b, b)

        plsc.subcore_barrier()
        pltpu.sync_copy(agg.at[pl.ds(sid * RPT, RPT)],
                        out_hbm.at[pl.ds(cid * N + sid * RPT, RPT)])

    return k(feat, src3, dst3, zeros)


def _dot_t(a, w):
    return lax.dot_general(a, w, (((1,), (1,)), ((), ())),
                           precision=lax.Precision.HIGHEST,
                           preferred_element_type=jnp.float32)


def _tc1_body(x_ref, wrel_ref, wroot_ref, b_ref, xr_ref, xroot_ref):
    x = x_ref[...]
    xr_ref[...] = _dot_t(x, wrel_ref[...])
    xroot_ref[...] = _dot_t(x, wroot_ref[...]) + b_ref[...]


def _tc2_body(part_ref, xroot_ref, wrel_ref, wroot_ref, b_ref, hr_ref,
              hroot_ref):
    h = jnp.maximum(part_ref[0] + part_ref[1] + xroot_ref[...], 0.0)
    hr_ref[...] = _dot_t(h, wrel_ref[...])
    hroot_ref[...] = _dot_t(h, wroot_ref[...]) + b_ref[...]


def _tc3_body(part_ref, hroot_ref, out_ref):
    z = part_ref[0] + part_ref[1] + hroot_ref[...]
    zmax = jnp.max(z, axis=1, keepdims=True)
    s = z - zmax
    out_ref[...] = s - jnp.log(jnp.sum(jnp.exp(s), axis=1, keepdims=True))


def kernel(x, edge_index, W1_rel, b1, W1_root, W2_rel, b2, W2_root):
    pad_cnt = EPAD - E
    ar = jnp.arange(pad_cnt, dtype=jnp.int32)
    pad_src = (ar * 131) % N
    pad_dst = N + (ar % 16)
    src3 = jnp.concatenate([edge_index[0], pad_src]).reshape(NW, NCHUNK, CH)
    dst3 = jnp.concatenate([edge_index[1], pad_dst]).reshape(NW, NCHUNK, CH)
    zeros = jnp.zeros((N, H), jnp.float32)
    b1r = b1.reshape(1, H)
    b2r = b2.reshape(1, C)

    xr, xroot = pl.pallas_call(
        _tc1_body,
        out_shape=[jax.ShapeDtypeStruct((N, H), jnp.float32),
                   jax.ShapeDtypeStruct((N, H), jnp.float32)],
    )(x, W1_rel, W1_root, b1r)

    part1 = _seg_sum_sc(xr, src3, dst3, zeros).reshape(NC, N, H)

    hr, hroot = pl.pallas_call(
        _tc2_body,
        out_shape=[jax.ShapeDtypeStruct((N, C), jnp.float32),
                   jax.ShapeDtypeStruct((N, C), jnp.float32)],
    )(part1, xroot, W2_rel, W2_root, b2r)

    part2 = _seg_sum_sc(hr, src3, dst3, zeros).reshape(NC, N, C)

    out = pl.pallas_call(
        _tc3_body,
        out_shape=jax.ShapeDtypeStruct((N, C), jnp.float32),
    )(part2, hroot)

    return out
